# baseline (speedup 1.0000x reference)
import functools

import jax
import jax.numpy as jnp
from jax import lax
from jax.experimental import pallas as pl
from jax.experimental.pallas import tpu as pltpu
from jax.experimental.pallas import tpu_sc as plsc

_N_NODES = 100000
_N_EDGES = 6400000
_NW = 32
_CHUNK = 1280
_N_CH = _N_EDGES // _CHUNK
_VSTEPS = _CHUNK // 16


def _make_partial_loss():
    mesh = plsc.VectorSubcoreMesh(core_axis_name="c", subcore_axis_name="s")

    @functools.partial(
        pl.kernel,
        mesh=mesh,
        out_type=jax.ShapeDtypeStruct((_NW, 16), jnp.float32),
        compiler_params=pltpu.CompilerParams(needs_layout_passes=False),
        scratch_types=[
            pltpu.VMEM((_N_NODES,), jnp.int32),
            pltpu.VMEM((2, _CHUNK), jnp.int32),
            pltpu.VMEM((2, _CHUNK), jnp.int32),
            pltpu.VMEM((_CHUNK,), jnp.float32),
            pltpu.VMEM((_CHUNK,), jnp.float32),
            pltpu.VMEM((16,), jnp.float32),
            pltpu.SemaphoreType.DMA,
            pltpu.SemaphoreType.DMA,
            pltpu.SemaphoreType.DMA,
        ],
    )
    def partial_loss(pred_hbm, eidx_hbm, group_hbm, out_hbm,
                     group_v, eid0_v, eid1_v, pred0_v, pred1_v,
                     acc_v, sem0, sem1, semg):
        cid = lax.axis_index("c")
        sid = lax.axis_index("s")
        wid = sid * 2 + cid
        n_t = (_N_CH - wid + _NW - 1) // _NW
        sems = (sem0, sem1)
        eids = (eid0_v, eid1_v)
        preds = (pred0_v, pred1_v)

        pltpu.async_copy(group_hbm, group_v, semg)

        def start(t, b):
            off = (wid + _NW * t) * _CHUNK
            pltpu.async_copy(eidx_hbm.at[:, pl.ds(off, _CHUNK)],
                             eids[b], sems[b])
            pltpu.async_copy(pred_hbm.at[pl.ds(off, _CHUNK)],
                             preds[b], sems[b])

        def wait(b):
            pltpu.make_async_copy(eidx_hbm.at[:, pl.ds(0, _CHUNK)],
                                  eids[b], sems[b]).wait()
            pltpu.make_async_copy(pred_hbm.at[pl.ds(0, _CHUNK)],
                                  preds[b], sems[b]).wait()

        iota16 = jnp.arange(16, dtype=jnp.int32)
        row0 = jnp.zeros((16,), jnp.int32)
        row1 = jnp.ones((16,), jnp.int32)

        def compute(b, acc):
            eref, pref = eids[b], preds[b]

            def step(i, a):
                cols = i * 16 + iota16
                s = plsc.load_gather(eref, [row0, cols])
                t = plsc.load_gather(eref, [row1, cols])
                p = pref[pl.ds(i * 16, 16)]
                sg = plsc.load_gather(group_v, [s])
                tg = plsc.load_gather(group_v, [t])
                assn = jnp.where(sg == tg, jnp.float32(1.0),
                                 jnp.float32(0.0))
                d = assn - p
                return a + d * d

            return plsc.parallel_loop(0, _VSTEPS, unroll=8, carry=acc)(step)

        start(0, 0)
        start(1, 1)
        pltpu.make_async_copy(group_hbm, group_v, semg).wait()

        def pair_body(g, acc):
            c0 = 2 * g
            wait(0)

            @pl.when(c0 + 2 < n_t)
            def _():
                start(c0 + 2, 0)

            acc = compute(0, acc)
            wait(1)

            @pl.when(c0 + 3 < n_t)
            def _():
                start(c0 + 3, 1)

            return compute(1, acc)

        acc = lax.fori_loop(0, n_t // 2, pair_body,
                            jnp.zeros((16,), jnp.float32))
        acc_v[...] = acc

        @pl.when(n_t % 2 == 1)
        def _():
            wait(0)
            acc_v[...] = compute(0, acc_v[...])

        pltpu.sync_copy(acc_v, out_hbm.at[wid])

    return partial_loss


_partial_loss = _make_partial_loss()


def kernel(edge_pred, edge_index, group):
    partials = _partial_loss(edge_pred, edge_index, group)
    return jnp.sum(partials) / jnp.float32(_N_EDGES)


# speedup vs baseline: 1.0188x; 1.0188x over previous
import functools

import jax
import jax.numpy as jnp
from jax import lax
from jax.experimental import pallas as pl
from jax.experimental.pallas import tpu as pltpu
from jax.experimental.pallas import tpu_sc as plsc

_N_NODES = 100000
_N_EDGES = 6400000
_NW = 32
_CHUNK = 3200
_N_CH = _N_EDGES // _CHUNK
_VSTEPS = _CHUNK // 16


def _make_partial_loss():
    mesh = plsc.VectorSubcoreMesh(core_axis_name="c", subcore_axis_name="s")

    @functools.partial(
        pl.kernel,
        mesh=mesh,
        out_type=jax.ShapeDtypeStruct((_NW, 16), jnp.float32),
        compiler_params=pltpu.CompilerParams(needs_layout_passes=False),
        scratch_types=[
            pltpu.VMEM((_N_NODES,), jnp.int32),
            pltpu.VMEM((2, _CHUNK), jnp.int32),
            pltpu.VMEM((2, _CHUNK), jnp.int32),
            pltpu.VMEM((_CHUNK,), jnp.float32),
            pltpu.VMEM((_CHUNK,), jnp.float32),
            pltpu.VMEM((16,), jnp.float32),
            pltpu.SemaphoreType.DMA,
            pltpu.SemaphoreType.DMA,
            pltpu.SemaphoreType.DMA,
        ],
    )
    def partial_loss(pred_hbm, eidx_hbm, group_hbm, out_hbm,
                     group_v, eid0_v, eid1_v, pred0_v, pred1_v,
                     acc_v, sem0, sem1, semg):
        cid = lax.axis_index("c")
        sid = lax.axis_index("s")
        wid = sid * 2 + cid
        n_t = (_N_CH - wid + _NW - 1) // _NW
        sems = (sem0, sem1)
        eids = (eid0_v, eid1_v)
        preds = (pred0_v, pred1_v)

        pltpu.async_copy(group_hbm, group_v, semg)

        def start(t, b):
            off = (wid + _NW * t) * _CHUNK
            pltpu.async_copy(eidx_hbm.at[:, pl.ds(off, _CHUNK)],
                             eids[b], sems[b])
            pltpu.async_copy(pred_hbm.at[pl.ds(off, _CHUNK)],
                             preds[b], sems[b])

        def wait(b):
            pltpu.make_async_copy(eidx_hbm.at[:, pl.ds(0, _CHUNK)],
                                  eids[b], sems[b]).wait()
            pltpu.make_async_copy(pred_hbm.at[pl.ds(0, _CHUNK)],
                                  preds[b], sems[b]).wait()

        iota16 = jnp.arange(16, dtype=jnp.int32)
        row0 = jnp.zeros((16,), jnp.int32)
        row1 = jnp.ones((16,), jnp.int32)

        def compute(b, acc):
            eref, pref = eids[b], preds[b]

            def step(i, a):
                cols = i * 16 + iota16
                s = plsc.load_gather(eref, [row0, cols])
                t = plsc.load_gather(eref, [row1, cols])
                p = pref[pl.ds(i * 16, 16)]
                sg = plsc.load_gather(group_v, [s])
                tg = plsc.load_gather(group_v, [t])
                assn = jnp.where(sg == tg, jnp.float32(1.0),
                                 jnp.float32(0.0))
                d = assn - p
                return a + d * d

            return plsc.parallel_loop(0, _VSTEPS, unroll=20, carry=acc)(step)

        start(0, 0)
        start(1, 1)
        pltpu.make_async_copy(group_hbm, group_v, semg).wait()

        def pair_body(g, acc):
            c0 = 2 * g
            wait(0)

            @pl.when(c0 + 2 < n_t)
            def _():
                start(c0 + 2, 0)

            acc = compute(0, acc)
            wait(1)

            @pl.when(c0 + 3 < n_t)
            def _():
                start(c0 + 3, 1)

            return compute(1, acc)

        acc = lax.fori_loop(0, n_t // 2, pair_body,
                            jnp.zeros((16,), jnp.float32))
        acc_v[...] = acc

        @pl.when(n_t % 2 == 1)
        def _():
            wait(0)
            acc_v[...] = compute(0, acc_v[...])

        pltpu.sync_copy(acc_v, out_hbm.at[wid])

    return partial_loss


_partial_loss = _make_partial_loss()


def kernel(edge_pred, edge_index, group):
    partials = _partial_loss(edge_pred, edge_index, group)
    return jnp.sum(partials) / jnp.float32(_N_EDGES)


# speedup vs baseline: 1.2607x; 1.2375x over previous
import functools

import jax
import jax.numpy as jnp
from jax import lax
from jax.experimental import pallas as pl
from jax.experimental.pallas import tpu as pltpu
from jax.experimental.pallas import tpu_sc as plsc

_N_NODES = 100000
_N_EDGES = 6400000
_NW = 32
_CHUNK = 3200
_N_CH = _N_EDGES // _CHUNK
_VSTEPS = _CHUNK // 16


def _make_partial_loss():
    mesh = plsc.VectorSubcoreMesh(core_axis_name="c", subcore_axis_name="s")

    @functools.partial(
        pl.kernel,
        mesh=mesh,
        out_type=jax.ShapeDtypeStruct((_NW, 16), jnp.float32),
        compiler_params=pltpu.CompilerParams(needs_layout_passes=False),
        scratch_types=[
            pltpu.VMEM((_N_NODES,), jnp.int32),
            pltpu.VMEM((2, _CHUNK), jnp.int32),
            pltpu.VMEM((2, _CHUNK), jnp.int32),
            pltpu.VMEM((_CHUNK,), jnp.float32),
            pltpu.VMEM((_CHUNK,), jnp.float32),
            pltpu.VMEM((16,), jnp.float32),
            pltpu.SemaphoreType.DMA,
            pltpu.SemaphoreType.DMA,
            pltpu.SemaphoreType.DMA,
        ],
    )
    def partial_loss(pred_hbm, eidx_hbm, group_hbm, out_hbm,
                     group_v, eid0_v, eid1_v, pred0_v, pred1_v,
                     acc_v, sem0, sem1, semg):
        cid = lax.axis_index("c")
        sid = lax.axis_index("s")
        wid = sid * 2 + cid
        n_t = (_N_CH - wid + _NW - 1) // _NW
        sems = (sem0, sem1)
        eids = (eid0_v, eid1_v)
        preds = (pred0_v, pred1_v)

        pltpu.async_copy(group_hbm, group_v, semg)

        def start(t, b):
            off = (wid + _NW * t) * _CHUNK
            pltpu.async_copy(eidx_hbm.at[:, pl.ds(off, _CHUNK)],
                             eids[b], sems[b])
            pltpu.async_copy(pred_hbm.at[pl.ds(off, _CHUNK)],
                             preds[b], sems[b])

        def wait(b):
            pltpu.make_async_copy(eidx_hbm.at[:, pl.ds(0, _CHUNK)],
                                  eids[b], sems[b]).wait()
            pltpu.make_async_copy(pred_hbm.at[pl.ds(0, _CHUNK)],
                                  preds[b], sems[b]).wait()

        iota16 = jnp.arange(16, dtype=jnp.int32)
        row0 = jnp.zeros((16,), jnp.int32)
        row1 = jnp.ones((16,), jnp.int32)

        def compute(b, acc):
            eref, pref = eids[b], preds[b]

            def step(i, a):
                cols = i * 16 + iota16
                s = plsc.load_gather(eref, [row0, cols])
                t = plsc.load_gather(eref, [row1, cols])
                p = pref[pl.ds(i * 16, 16)]
                sg = plsc.load_gather(group_v, [s])
                tg = plsc.load_gather(group_v, [t])
                assn = jnp.where(sg == tg, jnp.float32(1.0),
                                 jnp.float32(0.0))
                d = assn - p
                return a + d * d

            return plsc.parallel_loop(0, _VSTEPS, unroll=10, carry=acc)(step)

        start(0, 0)
        start(1, 1)
        pltpu.make_async_copy(group_hbm, group_v, semg).wait()

        def pair_body(g, acc):
            c0 = 2 * g
            wait(0)

            @pl.when(c0 + 2 < n_t)
            def _():
                start(c0 + 2, 0)

            acc = compute(0, acc)
            wait(1)

            @pl.when(c0 + 3 < n_t)
            def _():
                start(c0 + 3, 1)

            return compute(1, acc)

        acc = lax.fori_loop(0, n_t // 2, pair_body,
                            jnp.zeros((16,), jnp.float32))
        acc_v[...] = acc

        @pl.when(n_t % 2 == 1)
        def _():
            wait(0)
            acc_v[...] = compute(0, acc_v[...])

        pltpu.sync_copy(acc_v, out_hbm.at[wid])

    return partial_loss


_partial_loss = _make_partial_loss()


def kernel(edge_pred, edge_index, group):
    partials = _partial_loss(edge_pred, edge_index, group)
    return jnp.sum(partials) / jnp.float32(_N_EDGES)


# speedup vs baseline: 1.3690x; 1.0859x over previous
import functools

import jax
import jax.numpy as jnp
from jax import lax
from jax.experimental import pallas as pl
from jax.experimental.pallas import tpu as pltpu
from jax.experimental.pallas import tpu_sc as plsc

_N_NODES = 100000
_N_EDGES = 6400000
_NW = 32
_CHUNK = 3200
_N_CH = _N_EDGES // _CHUNK
_VSTEPS = _CHUNK // 16


def _make_partial_loss():
    mesh = plsc.VectorSubcoreMesh(core_axis_name="c", subcore_axis_name="s")

    @functools.partial(
        pl.kernel,
        mesh=mesh,
        out_type=jax.ShapeDtypeStruct((_NW, 16), jnp.float32),
        compiler_params=pltpu.CompilerParams(needs_layout_passes=False),
        scratch_types=[
            pltpu.VMEM((_N_NODES,), jnp.int32),
            pltpu.VMEM((2, _CHUNK), jnp.int32),
            pltpu.VMEM((2, _CHUNK), jnp.int32),
            pltpu.VMEM((_CHUNK,), jnp.float32),
            pltpu.VMEM((_CHUNK,), jnp.float32),
            pltpu.VMEM((16,), jnp.float32),
            pltpu.SemaphoreType.DMA,
            pltpu.SemaphoreType.DMA,
            pltpu.SemaphoreType.DMA,
            pltpu.VMEM_SHARED((_N_NODES,), jnp.int32),
        ],
    )
    def partial_loss(pred_hbm, eidx_hbm, group_hbm, out_hbm,
                     group_v, eid0_v, eid1_v, pred0_v, pred1_v,
                     acc_v, sem0, sem1, semg, group_s):
        cid = lax.axis_index("c")
        sid = lax.axis_index("s")
        wid = sid * 2 + cid
        n_t = (_N_CH - wid + _NW - 1) // _NW
        sems = (sem0, sem1)
        eids = (eid0_v, eid1_v)
        preds = (pred0_v, pred1_v)

        def start(t, b):
            off = (wid + _NW * t) * _CHUNK
            pltpu.async_copy(eidx_hbm.at[:, pl.ds(off, _CHUNK)],
                             eids[b], sems[b])
            pltpu.async_copy(pred_hbm.at[pl.ds(off, _CHUNK)],
                             preds[b], sems[b])

        def wait(b):
            pltpu.make_async_copy(eidx_hbm.at[:, pl.ds(0, _CHUNK)],
                                  eids[b], sems[b]).wait()
            pltpu.make_async_copy(pred_hbm.at[pl.ds(0, _CHUNK)],
                                  preds[b], sems[b]).wait()

        iota16 = jnp.arange(16, dtype=jnp.int32)
        row0 = jnp.zeros((16,), jnp.int32)
        row1 = jnp.ones((16,), jnp.int32)

        def compute(b, acc):
            eref, pref = eids[b], preds[b]

            def step(i, a):
                cols = i * 16 + iota16
                s = plsc.load_gather(eref, [row0, cols])
                t = plsc.load_gather(eref, [row1, cols])
                p = pref[pl.ds(i * 16, 16)]
                sg = plsc.load_gather(group_v, [s])
                tg = plsc.load_gather(group_v, [t])
                assn = jnp.where(sg == tg, jnp.float32(1.0),
                                 jnp.float32(0.0))
                d = assn - p
                return a + d * d

            return plsc.parallel_loop(0, _VSTEPS, unroll=8, carry=acc)(step)

        start(0, 0)
        start(1, 1)

        @pl.when(sid == 0)
        def _():
            pltpu.sync_copy(group_hbm, group_s)

        plsc.subcore_barrier()
        pltpu.sync_copy(group_s, group_v)

        def pair_body(g, acc):
            c0 = 2 * g
            wait(0)

            @pl.when(c0 + 2 < n_t)
            def _():
                start(c0 + 2, 0)

            acc = compute(0, acc)
            wait(1)

            @pl.when(c0 + 3 < n_t)
            def _():
                start(c0 + 3, 1)

            return compute(1, acc)

        acc = lax.fori_loop(0, n_t // 2, pair_body,
                            jnp.zeros((16,), jnp.float32))
        acc_v[...] = acc

        @pl.when(n_t % 2 == 1)
        def _():
            wait(0)
            acc_v[...] = compute(0, acc_v[...])

        pltpu.sync_copy(acc_v, out_hbm.at[wid])

    return partial_loss


_partial_loss = _make_partial_loss()


def kernel(edge_pred, edge_index, group):
    partials = _partial_loss(edge_pred, edge_index, group)
    return jnp.sum(partials) / jnp.float32(_N_EDGES)
